# baseline (speedup 1.0000x reference)
import functools

import jax
import jax.numpy as jnp
from jax import lax
from jax.experimental import pallas as pl
from jax.experimental.pallas import tpu as pltpu
from jax.experimental.pallas import tpu_sc as plsc

NC = 2
NS = 16
NW = NC * NS
CH = 88


@functools.lru_cache(maxsize=None)
def _make_seg_sum(n, d, e):
    per_w = e // NW
    nch = -(-per_w // CH)
    per_w_pad = nch * CH
    rows_per_tile = -(-(n + NW) // (NS * 8)) * 8
    n_pad = rows_per_tile * NS

    mesh = plsc.VectorSubcoreMesh(core_axis_name="c", subcore_axis_name="s")

    @functools.partial(
        pl.kernel,
        out_type=jax.ShapeDtypeStruct((NC, n_pad, d), jnp.float32),
        mesh=mesh,
        scratch_types=[
            pltpu.VMEM((per_w_pad,), jnp.int32),
            pltpu.VMEM((nch, CH), jnp.int32),
            pltpu.VMEM((CH, d), jnp.float32),
            pltpu.VMEM((CH, d), jnp.float32),
            pltpu.VMEM_SHARED((n_pad, d), jnp.float32),
            pltpu.SemaphoreType.DMA,
            pltpu.SemaphoreType.DMA,
            pltpu.SemaphoreType.DMA,
            pltpu.SemaphoreType.DMA,
        ],
    )
    def seg_sum(x_hbm, src_hbm, dst_hbm, zeros_hbm, out_hbm,
                srcb, dstb, rows_a, rows_b, agg, sem_a, sem_b,
                sem_sa, sem_sb):
        c = lax.axis_index("c")
        s = lax.axis_index("s")
        wid = s * NC + c
        base_n = s * rows_per_tile

        def fire(j, buf, sem):
            pltpu.async_copy(x_hbm.at[srcb.at[pl.ds(j * CH, CH)]], buf, sem)

        def gwait(j, buf, sem):
            pltpu.make_async_copy(x_hbm.at[srcb.at[pl.ds(j * CH, CH)]],
                                  buf, sem).wait()

        def scat(j, buf, sem):
            pltpu.async_copy(buf, agg.at[dstb.at[j]], sem, add=True)

        def swait(j, buf, sem):
            pltpu.make_async_copy(buf, agg.at[dstb.at[j]], sem).wait()

        pltpu.async_copy(zeros_hbm.at[pl.ds(base_n, rows_per_tile)],
                         agg.at[pl.ds(base_n, rows_per_tile)], sem_a)
        pltpu.async_copy(src_hbm.at[wid], srcb, sem_b)
        pltpu.async_copy(dst_hbm.at[wid], dstb, sem_sa)
        pltpu.make_async_copy(zeros_hbm.at[pl.ds(base_n, rows_per_tile)],
                              agg.at[pl.ds(base_n, rows_per_tile)],
                              sem_a).wait()
        pltpu.make_async_copy(src_hbm.at[wid], srcb, sem_b).wait()
        pltpu.make_async_copy(dst_hbm.at[wid], dstb, sem_sa).wait()
        plsc.subcore_barrier()

        def drain(j, buf, gsem, ssem):
            gwait(j, buf, gsem)
            scat(j, buf, ssem)
            swait(j, buf, ssem)

        fire(0, rows_a, sem_a)
        if nch % 2 == 1:
            @pl.loop(0, nch - 1, step=2)
            def _(j):
                fire(j + 1, rows_b, sem_b)
                drain(j, rows_a, sem_a, sem_sa)
                fire(j + 2, rows_a, sem_a)
                drain(j + 1, rows_b, sem_b, sem_sb)

            drain(nch - 1, rows_a, sem_a, sem_sa)
        else:
            @pl.loop(0, nch, step=2)
            def _(j):
                fire(j + 1, rows_b, sem_b)
                drain(j, rows_a, sem_a, sem_sa)

                @pl.when(j + 2 < nch)
                def _():
                    fire(j + 2, rows_a, sem_a)

                drain(j + 1, rows_b, sem_b, sem_sb)

        plsc.subcore_barrier()
        pltpu.sync_copy(agg.at[pl.ds(base_n, rows_per_tile)],
                        out_hbm.at[c].at[pl.ds(base_n, rows_per_tile)])

    return seg_sum, nch, per_w, per_w_pad, n_pad


@functools.lru_cache(maxsize=None)
def _make_dense_root(n, d_in, d_out):
    blk = 1000
    grid = (n // blk,)

    def body(x_ref, wo_ref, b_ref, o_ref):
        o_ref[...] = jnp.dot(x_ref[...], wo_ref[...],
                             preferred_element_type=jnp.float32) + b_ref[...]

    return pl.pallas_call(
        body,
        grid=grid,
        in_specs=[
            pl.BlockSpec((blk, d_in), lambda i: (i, 0)),
            pl.BlockSpec((d_in, d_out), lambda i: (0, 0)),
            pl.BlockSpec((1, d_out), lambda i: (0, 0)),
        ],
        out_specs=pl.BlockSpec((blk, d_out), lambda i: (i, 0)),
        out_shape=jax.ShapeDtypeStruct((n, d_out), jnp.float32),
    )


@functools.lru_cache(maxsize=None)
def _make_dense_rel(n, d_in, d_out, relu):
    blk = 1000
    grid = (n // blk,)

    def body(a0_ref, a1_ref, r_ref, wr_ref, o_ref):
        a = a0_ref[...] + a1_ref[...]
        acc = jnp.dot(a, wr_ref[...], preferred_element_type=jnp.float32)
        acc = acc + r_ref[...]
        if relu:
            acc = jnp.maximum(acc, 0.0)
        o_ref[...] = acc

    return pl.pallas_call(
        body,
        grid=grid,
        in_specs=[
            pl.BlockSpec((blk, d_in), lambda i: (i, 0)),
            pl.BlockSpec((blk, d_in), lambda i: (i, 0)),
            pl.BlockSpec((blk, d_out), lambda i: (i, 0)),
            pl.BlockSpec((d_in, d_out), lambda i: (0, 0)),
        ],
        out_specs=pl.BlockSpec((blk, d_out), lambda i: (i, 0)),
        out_shape=jax.ShapeDtypeStruct((n, d_out), jnp.float32),
    )


def kernel(x, edge_index, W1_rel, b1, W1_root, W2_rel, b2, W2_root):
    n, d = x.shape
    e = edge_index.shape[1]
    seg_sum, nch, per_w, per_w_pad, n_pad = _make_seg_sum(n, d, e)
    pad = per_w_pad - per_w
    src = jnp.pad(edge_index[0].reshape(NW, per_w), ((0, 0), (0, pad)))
    dump = jnp.broadcast_to(n + jnp.arange(NW, dtype=jnp.int32)[:, None],
                            (NW, pad))
    dst = jnp.concatenate([edge_index[1].reshape(NW, per_w), dump],
                          axis=1).reshape(NW, nch, CH)
    zeros = jnp.zeros((n_pad, d), jnp.float32)

    p1 = seg_sum(x, src, dst, zeros)
    root1 = _make_dense_root(n, d, W1_root.shape[0])(x, W1_root.T, b1[None, :])
    h = _make_dense_rel(n, d, W1_rel.shape[0], True)(
        p1[0], p1[1], root1, W1_rel.T)
    p2 = seg_sum(h, src, dst, zeros)
    root2 = _make_dense_root(n, d, W2_root.shape[0])(h, W2_root.T, b2[None, :])
    out = _make_dense_rel(n, d, W2_rel.shape[0], False)(
        p2[0], p2[1], root2, W2_rel.T)
    return out


# speedup vs baseline: 1.2241x; 1.2241x over previous
import functools

import jax
import jax.numpy as jnp
from jax import lax
from jax.experimental import pallas as pl
from jax.experimental.pallas import tpu as pltpu
from jax.experimental.pallas import tpu_sc as plsc

NC = 2
NS = 16
NW = NC * NS
CH = 80


@functools.lru_cache(maxsize=None)
def _make_seg_sum(n, d, e):
    per_w = e // NW
    nch = -(-per_w // CH)
    per_w_pad = nch * CH
    rows_per_tile = -(-(n + NW) // (NS * 8)) * 8
    n_pad = rows_per_tile * NS

    mesh = plsc.VectorSubcoreMesh(core_axis_name="c", subcore_axis_name="s")

    @functools.partial(
        pl.kernel,
        out_type=jax.ShapeDtypeStruct((NC, n_pad, d), jnp.float32),
        mesh=mesh,
        scratch_types=[
            pltpu.VMEM((per_w_pad,), jnp.int32),
            pltpu.VMEM((nch, CH), jnp.int32),
            pltpu.VMEM((CH, d), jnp.float32),
            pltpu.VMEM((CH, d), jnp.float32),
            pltpu.VMEM_SHARED((n_pad, d), jnp.float32),
            pltpu.SemaphoreType.DMA,
            pltpu.SemaphoreType.DMA,
            pltpu.SemaphoreType.DMA,
            pltpu.SemaphoreType.DMA,
        ],
    )
    def seg_sum(x_hbm, src_hbm, dst_hbm, zeros_hbm, out_hbm,
                srcb, dstb, rows_a, rows_b, agg, sem_a, sem_b,
                sem_sa, sem_sb):
        c = lax.axis_index("c")
        s = lax.axis_index("s")
        wid = s * NC + c
        base_n = s * rows_per_tile

        def fire(j, buf, sem):
            pltpu.async_copy(x_hbm.at[srcb.at[pl.ds(j * CH, CH)]], buf, sem)

        def gwait(j, buf, sem):
            pltpu.make_async_copy(x_hbm.at[srcb.at[pl.ds(j * CH, CH)]],
                                  buf, sem).wait()

        def scat(j, buf, sem):
            pltpu.async_copy(buf, agg.at[dstb.at[j]], sem, add=True)

        def swait(j, buf, sem):
            pltpu.make_async_copy(buf, agg.at[dstb.at[j]], sem).wait()

        pltpu.async_copy(zeros_hbm.at[pl.ds(base_n, rows_per_tile)],
                         agg.at[pl.ds(base_n, rows_per_tile)], sem_a)
        pltpu.async_copy(src_hbm.at[wid], srcb, sem_b)
        pltpu.async_copy(dst_hbm.at[wid], dstb, sem_sa)
        pltpu.make_async_copy(zeros_hbm.at[pl.ds(base_n, rows_per_tile)],
                              agg.at[pl.ds(base_n, rows_per_tile)],
                              sem_a).wait()
        pltpu.make_async_copy(src_hbm.at[wid], srcb, sem_b).wait()
        pltpu.make_async_copy(dst_hbm.at[wid], dstb, sem_sa).wait()
        plsc.subcore_barrier()

        def drain(j, buf, gsem, ssem):
            gwait(j, buf, gsem)
            scat(j, buf, ssem)
            swait(j, buf, ssem)

        fire(0, rows_a, sem_a)
        if nch % 2 == 1:
            @pl.loop(0, nch - 1, step=2)
            def _(j):
                fire(j + 1, rows_b, sem_b)
                drain(j, rows_a, sem_a, sem_sa)
                fire(j + 2, rows_a, sem_a)
                drain(j + 1, rows_b, sem_b, sem_sb)

            drain(nch - 1, rows_a, sem_a, sem_sa)
        else:
            @pl.loop(0, nch, step=2)
            def _(j):
                fire(j + 1, rows_b, sem_b)
                drain(j, rows_a, sem_a, sem_sa)

                @pl.when(j + 2 < nch)
                def _():
                    fire(j + 2, rows_a, sem_a)

                drain(j + 1, rows_b, sem_b, sem_sb)

        plsc.subcore_barrier()
        pltpu.sync_copy(agg.at[pl.ds(base_n, rows_per_tile)],
                        out_hbm.at[c].at[pl.ds(base_n, rows_per_tile)])

    return seg_sum, nch, per_w, per_w_pad, n_pad


@functools.lru_cache(maxsize=None)
def _make_dense_root(n, d_in, d_out):
    blk = 2000
    grid = (n // blk,)

    def body(x_ref, wo_ref, b_ref, o_ref):
        o_ref[...] = jnp.dot(x_ref[...], wo_ref[...],
                             preferred_element_type=jnp.float32) + b_ref[...]

    return pl.pallas_call(
        body,
        grid=grid,
        in_specs=[
            pl.BlockSpec((blk, d_in), lambda i: (i, 0)),
            pl.BlockSpec((d_in, d_out), lambda i: (0, 0)),
            pl.BlockSpec((1, d_out), lambda i: (0, 0)),
        ],
        out_specs=pl.BlockSpec((blk, d_out), lambda i: (i, 0)),
        out_shape=jax.ShapeDtypeStruct((n, d_out), jnp.float32),
    )


@functools.lru_cache(maxsize=None)
def _make_dense_rel(n, d_in, d_out, relu):
    blk = 2000
    grid = (n // blk,)

    def body(a0_ref, a1_ref, r_ref, wr_ref, o_ref):
        a = a0_ref[...] + a1_ref[...]
        acc = jnp.dot(a, wr_ref[...], preferred_element_type=jnp.float32)
        acc = acc + r_ref[...]
        if relu:
            acc = jnp.maximum(acc, 0.0)
        o_ref[...] = acc

    return pl.pallas_call(
        body,
        grid=grid,
        in_specs=[
            pl.BlockSpec((blk, d_in), lambda i: (i, 0)),
            pl.BlockSpec((blk, d_in), lambda i: (i, 0)),
            pl.BlockSpec((blk, d_out), lambda i: (i, 0)),
            pl.BlockSpec((d_in, d_out), lambda i: (0, 0)),
        ],
        out_specs=pl.BlockSpec((blk, d_out), lambda i: (i, 0)),
        out_shape=jax.ShapeDtypeStruct((n, d_out), jnp.float32),
    )


def kernel(x, edge_index, W1_rel, b1, W1_root, W2_rel, b2, W2_root):
    n, d = x.shape
    e = edge_index.shape[1]
    seg_sum, nch, per_w, per_w_pad, n_pad = _make_seg_sum(n, d, e)
    pad = per_w_pad - per_w
    src = jnp.pad(edge_index[0].reshape(NW, per_w), ((0, 0), (0, pad)))
    dump = jnp.broadcast_to(n + jnp.arange(NW, dtype=jnp.int32)[:, None],
                            (NW, pad))
    dst = jnp.concatenate([edge_index[1].reshape(NW, per_w), dump],
                          axis=1).reshape(NW, nch, CH)
    zeros = jnp.zeros((n_pad, d), jnp.float32)

    p1 = seg_sum(x, src, dst, zeros)
    root1 = _make_dense_root(n, d, W1_root.shape[0])(x, W1_root.T, b1[None, :])
    h = _make_dense_rel(n, d, W1_rel.shape[0], True)(
        p1[0], p1[1], root1, W1_rel.T)
    p2 = seg_sum(h, src, dst, zeros)
    root2 = _make_dense_root(n, d, W2_root.shape[0])(h, W2_root.T, b2[None, :])
    out = _make_dense_rel(n, d, W2_rel.shape[0], False)(
        p2[0], p2[1], root2, W2_rel.T)
    return out
